# baseline (device time: 205698 ns/iter reference)
import jax
import jax.numpy as jnp
from jax import lax
from jax.experimental import pallas as pl
from jax.experimental.pallas import tpu as pltpu

N = 8
B = 256
D = 256
H = 512
N_LAYERS = 3


def _ring(k):
    return jnp.where(k < 4, k, 11 - k)


def kernel(x, Win0, Wout0, Win1, Wout1, Win2, Wout2):
    def body(x_ref, win0, wout0, win1, wout1, win2, wout2,
             out_ref, xbuf, pbuf, rbuf, send_sems, recv_sems, credit):
        my = lax.axis_index("i")
        r = _ring(my)
        right = _ring((r + 1) % N)
        left = _ring((r - 1) % N)

        barrier = pltpu.get_barrier_semaphore()
        for nbr in (left, right):
            pl.semaphore_signal(barrier, inc=1, device_id=(nbr,),
                                device_id_type=pl.DeviceIdType.MESH)
        pl.semaphore_wait(barrier, 2)

        def wait_credit():
            pl.semaphore_wait(credit, 1)

        def send_credit():
            pl.semaphore_signal(credit, inc=1, device_id=(left,),
                                device_id_type=pl.DeviceIdType.MESH)

        def chunk(ref, q):
            return ref.at[pl.ds(q * B, B), :]

        def all_gather():
            for h in range(N - 1):
                qs = (r - h) % N
                rdma = pltpu.make_async_remote_copy(
                    src_ref=chunk(xbuf, qs),
                    dst_ref=chunk(xbuf, qs),
                    send_sem=send_sems.at[h],
                    recv_sem=recv_sems.at[h],
                    device_id=(right,),
                    device_id_type=pl.DeviceIdType.MESH,
                )
                rdma.start()
                rdma.wait()

        def compute(win, wout):
            for q in range(N):
                xq = xbuf[q * B:(q + 1) * B, :]
                hq = jnp.maximum(
                    jnp.dot(xq, win[...], preferred_element_type=jnp.float32),
                    0.0,
                )
                pbuf[q * B:(q + 1) * B, :] = jnp.dot(
                    hq, wout[...], preferred_element_type=jnp.float32
                )

        def reduce_scatter():
            for h in range(N - 1):
                qs = (r - 1 - h) % N
                qr = (r - 2 - h) % N
                rdma = pltpu.make_async_remote_copy(
                    src_ref=chunk(pbuf, qs),
                    dst_ref=rbuf.at[h],
                    send_sem=send_sems.at[h],
                    recv_sem=recv_sems.at[h],
                    device_id=(right,),
                    device_id_type=pl.DeviceIdType.MESH,
                )
                rdma.start()
                rdma.wait()
                dst = chunk(pbuf, qr)
                dst[...] = dst[...] + rbuf[h, :, :]

        chunk(xbuf, r)[...] = x_ref[...]
        weights = [(win0, wout0), (win1, wout1), (win2, wout2)]
        for l, (win, wout) in enumerate(weights):
            if l > 0:
                wait_credit()
            all_gather()
            send_credit()
            compute(win, wout)
            wait_credit()
            reduce_scatter()
            if l < N_LAYERS - 1:
                send_credit()
                chunk(xbuf, r)[...] = chunk(pbuf, r)[...]

        out_ref[...] = chunk(pbuf, r)[...]

    return pl.pallas_call(
        body,
        out_shape=jax.ShapeDtypeStruct((B, D), jnp.float32),
        in_specs=[pl.BlockSpec(memory_space=pltpu.VMEM)] * 7,
        out_specs=pl.BlockSpec(memory_space=pltpu.VMEM),
        scratch_shapes=[
            pltpu.VMEM((N * B, D), jnp.float32),
            pltpu.VMEM((N * B, D), jnp.float32),
            pltpu.VMEM((N - 1, B, D), jnp.float32),
            pltpu.SemaphoreType.DMA((N - 1,)),
            pltpu.SemaphoreType.DMA((N - 1,)),
            pltpu.SemaphoreType.REGULAR,
        ],
        compiler_params=pltpu.CompilerParams(collective_id=0),
    )(x, Win0, Wout0, Win1, Wout1, Win2, Wout2)


# device time: 164528 ns/iter; 1.2502x vs baseline; 1.2502x over previous
import jax
import jax.numpy as jnp
from jax import lax
from jax.experimental import pallas as pl
from jax.experimental.pallas import tpu as pltpu

N = 8
B = 256
D = 256
H = 512
N_LAYERS = 3

AGX, AGY, AGZ, RSZ, RSY, RSX = range(6)


def kernel(x, Win0, Wout0, Win1, Wout1, Win2, Wout2):
    def body(x_ref, win0, wout0, win1, wout1, win2, wout2,
             out_ref, xbuf, pbuf, rbuf, send_sems, recv_sems):
        p = lax.axis_index("i")
        cz = p // 4
        p2 = p % 4
        ybit = p2 // 2
        px = jnp.bitwise_xor(p, 1)
        py = 4 * cz + (3 - p2)
        pz = jnp.bitwise_xor(p, 4)

        c_pair = 4 * cz + 2 * ybit
        c_half = 4 * cz
        o_half = 4 * (1 - cz)
        o_pair = 4 * cz + 2 * (1 - ybit)

        barrier = pltpu.get_barrier_semaphore()
        for nbr in (px, py, pz):
            pl.semaphore_signal(barrier, inc=1, device_id=(nbr,),
                                device_id_type=pl.DeviceIdType.MESH)
        pl.semaphore_wait(barrier, 3)

        def rows(ref, c, n):
            return ref.at[pl.ds(c * B, n * B), :]

        def exchange(src, dst, sem, partner):
            rdma = pltpu.make_async_remote_copy(
                src_ref=src, dst_ref=dst,
                send_sem=send_sems.at[sem], recv_sem=recv_sems.at[sem],
                device_id=(partner,), device_id_type=pl.DeviceIdType.MESH,
            )
            rdma.start()
            rdma.wait()

        def all_gather():
            exchange(rows(xbuf, p, 1), rows(xbuf, p, 1), AGX, px)
            exchange(rows(xbuf, c_pair, 2), rows(xbuf, c_pair, 2), AGY, py)
            exchange(rows(xbuf, c_half, 4), rows(xbuf, c_half, 4), AGZ, pz)

        def compute(win, wout):
            for q in range(N):
                xq = xbuf[q * B:(q + 1) * B, :]
                hq = jnp.maximum(
                    jnp.dot(xq, win[...], preferred_element_type=jnp.float32),
                    0.0,
                )
                pbuf[q * B:(q + 1) * B, :] = jnp.dot(
                    hq, wout[...], preferred_element_type=jnp.float32
                )

        def reduce_scatter():
            exchange(rows(pbuf, o_half, 4), rbuf.at[pl.ds(0, 4 * B), :],
                     RSZ, pz)
            acc = rows(pbuf, c_half, 4)
            acc[...] = acc[...] + rbuf[0:4 * B, :]
            exchange(rows(pbuf, o_pair, 2), rbuf.at[pl.ds(4 * B, 2 * B), :],
                     RSY, py)
            acc = rows(pbuf, c_pair, 2)
            acc[...] = acc[...] + rbuf[4 * B:6 * B, :]
            exchange(rows(pbuf, px, 1), rbuf.at[pl.ds(6 * B, B), :],
                     RSX, px)
            acc = rows(pbuf, p, 1)
            acc[...] = acc[...] + rbuf[6 * B:7 * B, :]

        rows(xbuf, p, 1)[...] = x_ref[...]
        weights = [(win0, wout0), (win1, wout1), (win2, wout2)]
        for l, (win, wout) in enumerate(weights):
            all_gather()
            compute(win, wout)
            reduce_scatter()
            if l < N_LAYERS - 1:
                rows(xbuf, p, 1)[...] = rows(pbuf, p, 1)[...]

        out_ref[...] = rows(pbuf, p, 1)[...]

    return pl.pallas_call(
        body,
        out_shape=jax.ShapeDtypeStruct((B, D), jnp.float32),
        in_specs=[pl.BlockSpec(memory_space=pltpu.VMEM)] * 7,
        out_specs=pl.BlockSpec(memory_space=pltpu.VMEM),
        scratch_shapes=[
            pltpu.VMEM((N * B, D), jnp.float32),
            pltpu.VMEM((N * B, D), jnp.float32),
            pltpu.VMEM(((N - 1) * B, D), jnp.float32),
            pltpu.SemaphoreType.DMA((6,)),
            pltpu.SemaphoreType.DMA((6,)),
        ],
        compiler_params=pltpu.CompilerParams(collective_id=0),
    )(x, Win0, Wout0, Win1, Wout1, Win2, Wout2)


# device time: 117081 ns/iter; 1.7569x vs baseline; 1.4052x over previous
import jax
import jax.numpy as jnp
from jax import lax
from jax.experimental import pallas as pl
from jax.experimental.pallas import tpu as pltpu

N = 8
B = 256
D = 256
H = 512
N_LAYERS = 3

AX0, AY0, AY1, AZ0, AZ1, AZ2, AZ3, RZ0, RZ1, RZ2, RZ3, RY0, RY1, RX0 = range(14)


def kernel(x, Win0, Wout0, Win1, Wout1, Win2, Wout2):
    def body(x_ref, win0, wout0, win1, wout1, win2, wout2,
             out_ref, xbuf, pbuf, rbuf, send_sems, recv_sems):
        p = lax.axis_index("i")
        px = jnp.bitwise_xor(p, 1)
        py = jnp.bitwise_xor(p, 3)
        pxy = jnp.bitwise_xor(p, 2)
        pz = jnp.bitwise_xor(p, 4)

        barrier = pltpu.get_barrier_semaphore()
        for nbr in (px, py, pz):
            pl.semaphore_signal(barrier, inc=1, device_id=(nbr,),
                                device_id_type=pl.DeviceIdType.MESH)
        pl.semaphore_wait(barrier, 3)

        def chunk(ref, c):
            return ref.at[pl.ds(c * B, B), :]

        def piece(src, dst, sem, partner):
            rdma = pltpu.make_async_remote_copy(
                src_ref=src, dst_ref=dst,
                send_sem=send_sems.at[sem], recv_sem=recv_sems.at[sem],
                device_id=(partner,), device_id_type=pl.DeviceIdType.MESH,
            )
            rdma.start()
            return rdma

        def mlp_chunk(c, win, wout):
            xq = xbuf[pl.ds(c * B, B), :]
            hq = jnp.maximum(
                jnp.dot(xq, win[...], preferred_element_type=jnp.float32), 0.0)
            pbuf[pl.ds(c * B, B), :] = jnp.dot(
                hq, wout[...], preferred_element_type=jnp.float32)

        def acc(c, slot):
            tgt = chunk(pbuf, c)
            tgt[...] = tgt[...] + rbuf[slot * B:(slot + 1) * B, :]

        weights = [(win0, wout0), (win1, wout1), (win2, wout2)]
        chunk(xbuf, p)[...] = x_ref[...]
        for l, (win, wout) in enumerate(weights):
            ax0 = piece(chunk(xbuf, p), chunk(xbuf, p), AX0, px)
            ay0 = piece(chunk(xbuf, p), chunk(xbuf, p), AY0, py)
            az0 = piece(chunk(xbuf, p), chunk(xbuf, p), AZ0, pz)
            mlp_chunk(p, win, wout)

            ax0.wait()
            ay1 = piece(chunk(xbuf, px), chunk(xbuf, px), AY1, py)
            az1 = piece(chunk(xbuf, px), chunk(xbuf, px), AZ1, pz)
            mlp_chunk(px, win, wout)

            ay0.wait()
            az2 = piece(chunk(xbuf, py), chunk(xbuf, py), AZ2, pz)
            ay1.wait()
            az3 = piece(chunk(xbuf, pxy), chunk(xbuf, pxy), AZ3, pz)
            mlp_chunk(py, win, wout)
            mlp_chunk(pxy, win, wout)

            zsrc = [pz, jnp.bitwise_xor(pz, 1),
                    jnp.bitwise_xor(pz, 3), jnp.bitwise_xor(pz, 2)]
            rz = []
            for k, az in enumerate([az0, az1, az2, az3]):
                az.wait()
                mlp_chunk(zsrc[k], win, wout)
                rz.append(piece(chunk(pbuf, zsrc[k]),
                                rbuf.at[pl.ds(k * B, B), :],
                                [RZ0, RZ1, RZ2, RZ3][k], pz))

            rz[2].wait()
            acc(py, 2)
            ry0 = piece(chunk(pbuf, py), rbuf.at[pl.ds(4 * B, B), :], RY0, py)
            rz[3].wait()
            acc(pxy, 3)
            ry1 = piece(chunk(pbuf, pxy), rbuf.at[pl.ds(5 * B, B), :], RY1, py)

            rz[1].wait()
            acc(px, 1)
            ry1.wait()
            acc(px, 5)
            rx0 = piece(chunk(pbuf, px), rbuf.at[pl.ds(6 * B, B), :], RX0, px)

            rz[0].wait()
            acc(p, 0)
            ry0.wait()
            acc(p, 4)
            rx0.wait()
            acc(p, 6)

            if l < N_LAYERS - 1:
                chunk(xbuf, p)[...] = chunk(pbuf, p)[...]

        out_ref[...] = chunk(pbuf, p)[...]

    return pl.pallas_call(
        body,
        out_shape=jax.ShapeDtypeStruct((B, D), jnp.float32),
        in_specs=[pl.BlockSpec(memory_space=pltpu.VMEM)] * 7,
        out_specs=pl.BlockSpec(memory_space=pltpu.VMEM),
        scratch_shapes=[
            pltpu.VMEM((N * B, D), jnp.float32),
            pltpu.VMEM((N * B, D), jnp.float32),
            pltpu.VMEM((7 * B, D), jnp.float32),
            pltpu.SemaphoreType.DMA((14,)),
            pltpu.SemaphoreType.DMA((14,)),
        ],
        compiler_params=pltpu.CompilerParams(collective_id=0),
    )(x, Win0, Wout0, Win1, Wout1, Win2, Wout2)
